# Initial kernel scaffold; baseline (speedup 1.0000x reference)
import jax
import jax.numpy as jnp
from jax.experimental import pallas as pl


def kernel(x, bin_values):
    raise NotImplementedError("write your pallas kernel here")


# speedup vs baseline: 2201.0918x; 2201.0918x over previous
import functools

import jax
import jax.numpy as jnp
from jax import lax
from jax.experimental import pallas as pl
from jax.experimental.pallas import tpu as pltpu
from jax.experimental.pallas import tpu_sc as plsc

N_BINS = 1024
N = 4_000_000
NC = 2
NS = 16
NW = NC * NS
PER_W = N // NW
CHUNK = 5000
NCHUNK = PER_W // CHUNK
FULL = CHUNK // 16
TAIL_OFF = CHUNK - 16


def _lookup(bins_v, xx):
    u = xx * jnp.float32(256.0)
    iu = u.astype(jnp.int32)
    uf = iu.astype(jnp.float32)
    idx = iu + jnp.where(uf > u, jnp.int32(511), jnp.int32(512))
    idx = jnp.clip(idx, jnp.int32(0), jnp.int32(N_BINS - 1))
    return plsc.load_gather(bins_v, [idx])


def _body(x_hbm, bins_hbm, out_hbm, bins_v, x_v, y_v):
    c = lax.axis_index("c")
    s = lax.axis_index("s")
    wid = s * NC + c
    base = wid * PER_W
    pltpu.sync_copy(bins_hbm, bins_v)

    def chunk_body(g, carry):
        off = base + g * CHUNK
        pltpu.sync_copy(x_hbm.at[pl.ds(off, CHUNK)], x_v)

        def vbody(i, c2):
            sl = pl.ds(i * 16, 16)
            y_v[sl] = _lookup(bins_v, x_v[sl])
            return c2

        lax.fori_loop(0, FULL, vbody, None)
        sl = pl.ds(TAIL_OFF, 16)
        y_v[sl] = _lookup(bins_v, x_v[sl])
        pltpu.sync_copy(y_v, out_hbm.at[pl.ds(off, CHUNK)])
        return carry

    lax.fori_loop(0, NCHUNK, chunk_body, None)


_pcn = functools.partial(
    pl.kernel,
    out_type=jax.ShapeDtypeStruct((N,), jnp.float32),
    mesh=plsc.VectorSubcoreMesh(
        core_axis_name="c", subcore_axis_name="s", num_cores=NC, num_subcores=NS
    ),
    scratch_types=[
        pltpu.VMEM((N_BINS,), jnp.float32),
        pltpu.VMEM((CHUNK,), jnp.float32),
        pltpu.VMEM((CHUNK,), jnp.float32),
    ],
    compiler_params=pltpu.CompilerParams(
        use_tc_tiling_on_sc=False, needs_layout_passes=False
    ),
)(_body)


@jax.jit
def kernel(x, bin_values):
    out = _pcn(x.reshape(-1), bin_values)
    return out[:, None]


# speedup vs baseline: 2589.8799x; 1.1766x over previous
import functools

import jax
import jax.numpy as jnp
from jax import lax
from jax.experimental import pallas as pl
from jax.experimental.pallas import tpu as pltpu
from jax.experimental.pallas import tpu_sc as plsc

N_BINS = 1024
N = 4_000_000
NC = 2
NS = 16
NW = NC * NS
PER_W = N // NW
CHUNK = 25_000
NCHUNK = PER_W // CHUNK
FULL_SPAN = (CHUNK // 16) * 16
TAIL_OFF = CHUNK - 16
UNROLL = 8


def _lookup(bins_v, xx):
    u = xx * jnp.float32(256.0)
    iu = u.astype(jnp.int32)
    uf = iu.astype(jnp.float32)
    idx = iu + jnp.where(uf > u, jnp.int32(511), jnp.int32(512))
    idx = jnp.clip(idx, jnp.int32(0), jnp.int32(N_BINS - 1))
    return plsc.load_gather(bins_v, [idx])


def _body(x_hbm, bins_hbm, out_hbm, bins_v, x_v0, x_v1, y_v0, y_v1,
          si0, si1, so0, so1):
    c = lax.axis_index("c")
    s = lax.axis_index("s")
    wid = s * NC + c
    base = wid * PER_W
    pltpu.sync_copy(bins_hbm, bins_v)

    x_bufs = [x_v0, x_v1]
    y_bufs = [y_v0, y_v1]
    sin = [si0, si1]
    sout = [so0, so1]
    in_d = [None, None]
    out_d = [None, None]

    def issue_in(g):
        b = g % 2
        in_d[b] = pltpu.async_copy(
            x_hbm.at[pl.ds(base + g * CHUNK, CHUNK)], x_bufs[b], sin[b])

    issue_in(0)
    for g in range(NCHUNK):
        b = g % 2
        if g + 1 < NCHUNK:
            issue_in(g + 1)
        in_d[b].wait()
        if out_d[b] is not None:
            out_d[b].wait()
        x_v = x_bufs[b]
        y_v = y_bufs[b]

        @plsc.parallel_loop(0, FULL_SPAN, 16, unroll=UNROLL)
        def vloop(i):
            sl = pl.ds(i, 16)
            y_v[sl] = _lookup(bins_v, x_v[sl])

        sl = pl.ds(TAIL_OFF, 16)
        y_v[sl] = _lookup(bins_v, x_v[sl])
        out_d[b] = pltpu.async_copy(
            y_bufs[b], out_hbm.at[pl.ds(base + g * CHUNK, CHUNK)], sout[b])

    out_d[(NCHUNK - 1) % 2].wait()
    out_d[NCHUNK % 2].wait()


_pcn = functools.partial(
    pl.kernel,
    out_type=jax.ShapeDtypeStruct((N,), jnp.float32),
    mesh=plsc.VectorSubcoreMesh(
        core_axis_name="c", subcore_axis_name="s", num_cores=NC, num_subcores=NS
    ),
    scratch_types=[
        pltpu.VMEM((N_BINS,), jnp.float32),
        pltpu.VMEM((CHUNK,), jnp.float32),
        pltpu.VMEM((CHUNK,), jnp.float32),
        pltpu.VMEM((CHUNK,), jnp.float32),
        pltpu.VMEM((CHUNK,), jnp.float32),
        pltpu.SemaphoreType.DMA,
        pltpu.SemaphoreType.DMA,
        pltpu.SemaphoreType.DMA,
        pltpu.SemaphoreType.DMA,
    ],
    compiler_params=pltpu.CompilerParams(
        use_tc_tiling_on_sc=False, needs_layout_passes=False
    ),
)(_body)


@jax.jit
def kernel(x, bin_values):
    out = _pcn(x.reshape(-1), bin_values)
    return out[:, None]


# speedup vs baseline: 4713.8383x; 1.8201x over previous
import functools

import jax
import jax.numpy as jnp
from jax import lax
from jax.experimental import pallas as pl
from jax.experimental.pallas import tpu as pltpu
from jax.experimental.pallas import tpu_sc as plsc

N_BINS = 1024
N = 4_000_000
K = 5
P = N // K
NC = 2
NS = 16
NW = NC * NS
PER_W = P // NW
CHUNK = 5_000
NCHUNK = PER_W // CHUNK
FULL_SPAN = (CHUNK // 16) * 16
TAIL_OFF = CHUNK - 16
UNROLL = 8


def _lookup(bins_v, xx):
    u = xx * jnp.float32(256.0)
    iu = u.astype(jnp.int32)
    uf = iu.astype(jnp.float32)
    idx = iu + jnp.where(uf > u, jnp.int32(511), jnp.int32(512))
    idx = jnp.clip(idx, jnp.int32(0), jnp.int32(N_BINS - 1))
    return plsc.load_gather(bins_v, [idx])


def _body(x_hbm, bins_hbm, out_hbm, bins_v, x_v0, x_v1, y_v0, y_v1,
          si0, si1, so0, so1):
    c = lax.axis_index("c")
    s = lax.axis_index("s")
    wid = s * NC + c
    base = wid * PER_W
    pltpu.sync_copy(bins_hbm, bins_v)

    x_bufs = [x_v0, x_v1]
    y_bufs = [y_v0, y_v1]
    sin = [si0, si1]
    sout = [so0, so1]
    in_d = [None, None]
    out_d = [None, None]

    def issue_in(g):
        b = g % 2
        in_d[b] = pltpu.async_copy(
            x_hbm.at[pl.ds(base + g * CHUNK, CHUNK)], x_bufs[b], sin[b])

    issue_in(0)
    for g in range(NCHUNK):
        b = g % 2
        if g + 1 < NCHUNK:
            issue_in(g + 1)
        in_d[b].wait()
        if out_d[b] is not None:
            out_d[b].wait()
        x_v = x_bufs[b]
        y_v = y_bufs[b]

        @plsc.parallel_loop(0, FULL_SPAN, 16, unroll=UNROLL)
        def vloop(i):
            sl = pl.ds(i, 16)
            y_v[sl] = _lookup(bins_v, x_v[sl])

        sl = pl.ds(TAIL_OFF, 16)
        y_v[sl] = _lookup(bins_v, x_v[sl])
        out_d[b] = pltpu.async_copy(
            y_bufs[b], out_hbm.at[pl.ds(base + g * CHUNK, CHUNK)], sout[b])

    out_d[(NCHUNK - 1) % 2].wait()
    out_d[NCHUNK % 2].wait()


_pcn = functools.partial(
    pl.kernel,
    out_type=jax.ShapeDtypeStruct((P,), jnp.float32),
    mesh=plsc.VectorSubcoreMesh(
        core_axis_name="c", subcore_axis_name="s", num_cores=NC, num_subcores=NS
    ),
    scratch_types=[
        pltpu.VMEM((N_BINS,), jnp.float32),
        pltpu.VMEM((CHUNK,), jnp.float32),
        pltpu.VMEM((CHUNK,), jnp.float32),
        pltpu.VMEM((CHUNK,), jnp.float32),
        pltpu.VMEM((CHUNK,), jnp.float32),
        pltpu.SemaphoreType.DMA,
        pltpu.SemaphoreType.DMA,
        pltpu.SemaphoreType.DMA,
        pltpu.SemaphoreType.DMA,
    ],
    compiler_params=pltpu.CompilerParams(
        use_tc_tiling_on_sc=False, needs_layout_passes=False
    ),
)(_body)


@jax.jit
def kernel(x, bin_values):
    outs = []
    for i in range(K):
        piece = x[i * P:(i + 1) * P].reshape(P)
        outs.append(_pcn(piece, bin_values)[:, None])
    return jnp.concatenate(outs, axis=0)


# speedup vs baseline: 4719.6516x; 1.0012x over previous
import functools

import jax
import jax.numpy as jnp
from jax import lax
from jax.experimental import pallas as pl
from jax.experimental.pallas import tpu as pltpu
from jax.experimental.pallas import tpu_sc as plsc

N_BINS = 1024
N = 4_000_000
K = 5
P = N // K
NC = 2
NS = 16
NW = NC * NS
PER_W = P // NW
CHUNK = 5_000
NCHUNK = PER_W // CHUNK
FULL_SPAN = (CHUNK // 16) * 16
TAIL_OFF = CHUNK - 16
UNROLL = 8


def _lookup(bins_v, xx):
    u = xx * jnp.float32(256.0)
    iu = u.astype(jnp.int32)
    uf = iu.astype(jnp.float32)
    idx = iu + jnp.where(uf > u, jnp.int32(511), jnp.int32(512))
    idx = jnp.clip(idx, jnp.int32(0), jnp.int32(N_BINS - 1))
    return plsc.load_gather(bins_v, [idx])


def _body(x_hbm, bins_hbm, out_hbm, bins_v, x_v0, x_v1, y_v0, y_v1,
          si0, si1, so0, so1):
    c = lax.axis_index("c")
    s = lax.axis_index("s")
    wid = s * NC + c
    base = wid * PER_W
    pltpu.sync_copy(bins_hbm, bins_v)

    x_bufs = [x_v0, x_v1]
    y_bufs = [y_v0, y_v1]
    sin = [si0, si1]
    sout = [so0, so1]
    in_d = [None, None]
    out_d = [None, None]

    def issue_in(g):
        b = g % 2
        in_d[b] = pltpu.async_copy(
            x_hbm.at[pl.ds(base + g * CHUNK, CHUNK)], x_bufs[b], sin[b])

    issue_in(0)
    for g in range(NCHUNK):
        b = g % 2
        if g + 1 < NCHUNK:
            issue_in(g + 1)
        in_d[b].wait()
        if out_d[b] is not None:
            out_d[b].wait()
        x_v = x_bufs[b]
        y_v = y_bufs[b]

        @plsc.parallel_loop(0, FULL_SPAN, 16, unroll=UNROLL)
        def vloop(i):
            sl = pl.ds(i, 16)
            y_v[sl] = _lookup(bins_v, x_v[sl])

        sl = pl.ds(TAIL_OFF, 16)
        y_v[sl] = _lookup(bins_v, x_v[sl])
        out_d[b] = pltpu.async_copy(
            y_bufs[b], out_hbm.at[pl.ds(base + g * CHUNK, CHUNK)], sout[b])

    out_d[(NCHUNK - 1) % 2].wait()
    out_d[NCHUNK % 2].wait()


_pcn = functools.partial(
    pl.kernel,
    out_type=jax.ShapeDtypeStruct((P,), jnp.float32),
    mesh=plsc.VectorSubcoreMesh(
        core_axis_name="c", subcore_axis_name="s", num_cores=NC, num_subcores=NS
    ),
    scratch_types=[
        pltpu.VMEM((N_BINS,), jnp.float32),
        pltpu.VMEM((CHUNK,), jnp.float32),
        pltpu.VMEM((CHUNK,), jnp.float32),
        pltpu.VMEM((CHUNK,), jnp.float32),
        pltpu.VMEM((CHUNK,), jnp.float32),
        pltpu.SemaphoreType.DMA,
        pltpu.SemaphoreType.DMA,
        pltpu.SemaphoreType.DMA,
        pltpu.SemaphoreType.DMA,
    ],
    compiler_params=pltpu.CompilerParams(
        use_tc_tiling_on_sc=False, needs_layout_passes=False
    ),
)(_body)


@jax.jit
def kernel(x, bin_values):
    outs = []
    for i in range(K):
        piece = x[i * P:(i + 1) * P].reshape(P)
        piece = lax.optimization_barrier(piece)
        outs.append(_pcn(piece, bin_values)[:, None])
    return jnp.concatenate(outs, axis=0)
